# Initial kernel scaffold; baseline (speedup 1.0000x reference)
import jax
import jax.numpy as jnp
from jax.experimental import pallas as pl


def kernel(x, path_mask, W_root, b_root, W_d1, b_d1, W_d2, b_d2):
    raise NotImplementedError("write your pallas kernel here")


# speedup vs baseline: 2.4574x; 2.4574x over previous
import functools

import jax
import jax.numpy as jnp
from jax.experimental import pallas as pl
from jax.experimental.pallas import tpu as pltpu

B = 8192
D = 1024
BLK = 256
C = B + 4 * BLK
NBLK = C // BLK


def _compose_d1_body(wr_ref, w1_ref, br_ref, b1_ref, t_ref, bt_ref):
    w1 = w1_ref[0]
    t_ref[0] = jnp.dot(wr_ref[...], w1, preferred_element_type=jnp.float32)
    bt_ref[0] = jnp.dot(br_ref[...], w1, preferred_element_type=jnp.float32) + b1_ref[0]


def _compose_d2_body(t_ref, w2_ref, bt_ref, b2_ref, wc_ref, bc_ref):
    w2 = w2_ref[0]
    wc_ref[0] = jnp.dot(t_ref[0], w2, preferred_element_type=jnp.float32)
    bc_ref[0] = jnp.dot(bt_ref[0], w2, preferred_element_type=jnp.float32) + b2_ref[0]


def _compose(W_root, b_root, W_d1, b_d1, W_d2, b_d2):
    br = b_root.reshape(1, D)
    b1 = b_d1.reshape(2, 1, D)
    b2 = b_d2.reshape(4, 1, D)
    T, bt = pl.pallas_call(
        _compose_d1_body,
        grid=(2,),
        in_specs=[
            pl.BlockSpec((D, D), lambda c: (0, 0)),
            pl.BlockSpec((1, D, D), lambda c: (c, 0, 0)),
            pl.BlockSpec((1, D), lambda c: (0, 0)),
            pl.BlockSpec((1, 1, D), lambda c: (c, 0, 0)),
        ],
        out_specs=[
            pl.BlockSpec((1, D, D), lambda c: (c, 0, 0)),
            pl.BlockSpec((1, 1, D), lambda c: (c, 0, 0)),
        ],
        out_shape=[
            jax.ShapeDtypeStruct((2, D, D), jnp.float32),
            jax.ShapeDtypeStruct((2, 1, D), jnp.float32),
        ],
    )(W_root, W_d1, br, b1)
    Wc, bc = pl.pallas_call(
        _compose_d2_body,
        grid=(4,),
        in_specs=[
            pl.BlockSpec((1, D, D), lambda e: (e // 2, 0, 0)),
            pl.BlockSpec((1, D, D), lambda e: (e, 0, 0)),
            pl.BlockSpec((1, 1, D), lambda e: (e // 2, 0, 0)),
            pl.BlockSpec((1, 1, D), lambda e: (e, 0, 0)),
        ],
        out_specs=[
            pl.BlockSpec((1, D, D), lambda e: (e, 0, 0)),
            pl.BlockSpec((1, 1, D), lambda e: (e, 0, 0)),
        ],
        out_shape=[
            jax.ShapeDtypeStruct((4, D, D), jnp.float32),
            jax.ShapeDtypeStruct((4, 1, D), jnp.float32),
        ],
    )(T, W_d2, bt, b2)
    return Wc, bc


def _routed_mm_body(off_ref, x_ref, wc_ref, bc_ref, o_ref):
    del off_ref
    o_ref[...] = (
        jnp.dot(x_ref[...], wc_ref[0], preferred_element_type=jnp.float32)
        + bc_ref[0]
    )


def _block_expert(b, off_ref):
    s = b * BLK
    return (
        (s >= off_ref[1]).astype(jnp.int32)
        + (s >= off_ref[2]).astype(jnp.int32)
        + (s >= off_ref[3]).astype(jnp.int32)
    )


def _routed_mm(off, x_sorted, Wc, bc):
    spec = pltpu.PrefetchScalarGridSpec(
        num_scalar_prefetch=1,
        grid=(NBLK,),
        in_specs=[
            pl.BlockSpec((BLK, D), lambda b, off_ref: (b, 0)),
            pl.BlockSpec((1, D, D), lambda b, off_ref: (_block_expert(b, off_ref), 0, 0)),
            pl.BlockSpec((1, 1, D), lambda b, off_ref: (_block_expert(b, off_ref), 0, 0)),
        ],
        out_specs=pl.BlockSpec((BLK, D), lambda b, off_ref: (b, 0)),
    )
    return pl.pallas_call(
        _routed_mm_body,
        grid_spec=spec,
        out_shape=jax.ShapeDtypeStruct((C, D), jnp.float32),
    )(off, x_sorted, Wc, bc)


def kernel(x, path_mask, W_root, b_root, W_d1, b_d1, W_d2, b_d2):
    Wc, bc = _compose(W_root, b_root, W_d1, b_d1, W_d2, b_d2)

    leaf = path_mask[:, 0] * 2 + path_mask[:, 1]
    oh = jax.nn.one_hot(leaf, 4, dtype=jnp.int32)
    counts = oh.sum(axis=0)
    padded = ((counts + BLK - 1) // BLK) * BLK
    off = jnp.concatenate([jnp.zeros((1,), jnp.int32),
                           jnp.cumsum(padded).astype(jnp.int32)])
    rank = jnp.take_along_axis(jnp.cumsum(oh, axis=0) - oh,
                               leaf[:, None], axis=1)[:, 0]
    pos = off[leaf] + rank
    src = jnp.zeros((C,), jnp.int32).at[pos].set(
        jnp.arange(B, dtype=jnp.int32))

    x_sorted = x[src]
    y_sorted = _routed_mm(off, x_sorted, Wc, bc)
    return y_sorted[pos]


# speedup vs baseline: 2.4733x; 1.0065x over previous
import functools

import jax
import jax.numpy as jnp
from jax import lax
from jax.experimental import pallas as pl
from jax.experimental.pallas import tpu as pltpu
from jax.experimental.pallas import tpu_sc as plsc

B = 8192
D = 1024
BLK = 256
C = B + 4 * BLK
NBLK = C // BLK

NC = 2
NS = 16
NW = NC * NS
L = 16

_SC_MESH = plsc.VectorSubcoreMesh(core_axis_name="c", subcore_axis_name="s")

_RT_CHUNK = B // NS
_RT_FILL = C // NS

_GX_PER_W = C // NW
_GX_CH = 48
_GX_N = _GX_PER_W // _GX_CH
_UG_PER_W = B // NW
_UG_CH = 32
_UG_N = _UG_PER_W // _UG_CH


def _iota16():
    return lax.iota(jnp.int32, L)


def _route_body(leaf_hbm, pos_hbm, src_hbm, offs_hbm,
                leaf_v, posq_v, tokq_v, zer_v, cnt_v, call_v, offs_v,
                csh, sem):
    cid = lax.axis_index("c")
    sid = lax.axis_index("s")

    @pl.when(cid == 0)
    def _work():
        base = sid * _RT_CHUNK
        pltpu.sync_copy(leaf_hbm.at[pl.ds(base, _RT_CHUNK)], leaf_v)
        lane = _iota16()

        acc = [jnp.zeros((L,), jnp.int32) for _ in range(4)]
        for j in range(_RT_CHUNK // L):
            lv = leaf_v[pl.ds(j * L, L)]
            for e in range(4):
                acc[e] += (lv == e).astype(jnp.int32)
        cv = jnp.zeros((L,), jnp.int32)
        for e in range(4):
            cv = jnp.where(lane == e, jnp.sum(acc[e]), cv)
        cnt_v[...] = cv
        pltpu.sync_copy(cnt_v, csh.at[pl.ds(sid * L, L)])

        for j in range(_RT_FILL // L):
            zer_v[pl.ds(j * L, L)] = jnp.zeros((L,), jnp.int32)
        pltpu.sync_copy(zer_v, src_hbm.at[pl.ds(sid * _RT_FILL, _RT_FILL)])

        plsc.subcore_barrier()

        pltpu.sync_copy(csh, call_v)
        cnt = [[None] * 4 for _ in range(NS)]
        for t in range(NS):
            vt = call_v[pl.ds(t * L, L)]
            for e in range(4):
                cnt[t][e] = jnp.sum(jnp.where(lane == e, vt, 0))
        starts = []
        off_e = jnp.int32(0)
        offs_vec = jnp.zeros((L,), jnp.int32)
        for e in range(4):
            tot = jnp.int32(0)
            before = jnp.int32(0)
            for t in range(NS):
                tot = tot + cnt[t][e]
                before = before + jnp.where(sid > t, cnt[t][e], 0)
            starts.append(off_e + before)
            pad = (tot + (BLK - 1)) & jnp.int32(-BLK)
            offs_vec = offs_vec + jnp.where(lane >= e + 1, pad, 0)
            off_e = off_e + pad

        @pl.when(sid == 0)
        def _write_offs():
            offs_v[...] = offs_vec
            pltpu.sync_copy(offs_v, offs_hbm)

        run = list(starts)
        for j in range(_RT_CHUNK // L):
            lv = leaf_v[pl.ds(j * L, L)]
            posv = jnp.zeros((L,), jnp.int32)
            for e in range(4):
                m = lv == e
                mi = m.astype(jnp.int32)
                posv = jnp.where(m, run[e] + lax.cumsum(mi) - 1, posv)
                run[e] = run[e] + jnp.sum(mi)
            q, r = divmod(j * L, 128)
            posq_v[q, pl.ds(r, L)] = posv
            tokq_v[q, pl.ds(r, L)] = lane + (base + j * L)

        for q in range(_RT_CHUNK // 128):
            pltpu.sync_copy(posq_v.at[q], pos_hbm.at[pl.ds(base + q * 128, 128)])
        for q in range(_RT_CHUNK // 128):
            pltpu.async_copy(tokq_v.at[q], src_hbm.at[posq_v.at[q]], sem).wait()


def _route(leaf):
    return pl.kernel(
        _route_body,
        out_type=[
            jax.ShapeDtypeStruct((B,), jnp.int32),
            jax.ShapeDtypeStruct((C,), jnp.int32),
            jax.ShapeDtypeStruct((16,), jnp.int32),
        ],
        mesh=_SC_MESH,
        compiler_params=pltpu.CompilerParams(needs_layout_passes=False),
        scratch_types=[
            pltpu.VMEM((_RT_CHUNK,), jnp.int32),
            pltpu.VMEM((_RT_CHUNK // 128, 128), jnp.int32),
            pltpu.VMEM((_RT_CHUNK // 128, 128), jnp.int32),
            pltpu.VMEM((_RT_FILL,), jnp.int32),
            pltpu.VMEM((L,), jnp.int32),
            pltpu.VMEM((NS * L,), jnp.int32),
            pltpu.VMEM((L,), jnp.int32),
            pltpu.VMEM_SHARED((NS * L,), jnp.int32),
            pltpu.SemaphoreType.DMA,
        ],
    )(leaf)


def _gatherx_body(x_hbm, src_hbm, xs_hbm, idx_v, rows_a, rows_b, sem_a, sem_b):
    wid = lax.axis_index("s") * NC + lax.axis_index("c")
    base = wid * _GX_PER_W
    for k in range(_GX_N):
        pltpu.sync_copy(src_hbm.at[pl.ds(base + k * _GX_CH, _GX_CH)],
                        idx_v.at[k])
    bufs = (rows_a, rows_b)
    sems = (sem_a, sem_b)
    handles = [None, None]
    handles[0] = pltpu.async_copy(x_hbm.at[idx_v.at[0]], bufs[0], sems[0])
    for k in range(_GX_N):
        if k + 1 < _GX_N:
            nxt = (k + 1) % 2
            handles[nxt] = pltpu.async_copy(
                x_hbm.at[idx_v.at[k + 1]], bufs[nxt], sems[nxt])
        handles[k % 2].wait()
        pltpu.sync_copy(bufs[k % 2],
                        xs_hbm.at[pl.ds(base + k * _GX_CH, _GX_CH)])


def _gatherx(x, src):
    return pl.kernel(
        _gatherx_body,
        out_type=jax.ShapeDtypeStruct((C, D), jnp.float32),
        mesh=_SC_MESH,
        scratch_types=[
            pltpu.VMEM((_GX_N, _GX_CH), jnp.int32),
            pltpu.VMEM((_GX_CH, D), jnp.float32),
            pltpu.VMEM((_GX_CH, D), jnp.float32),
            pltpu.SemaphoreType.DMA,
            pltpu.SemaphoreType.DMA,
        ],
    )(x, src)


def _ungather_body(y_hbm, pos_hbm, out_hbm, idx_v, rows_a, rows_b, sem_a, sem_b):
    wid = lax.axis_index("s") * NC + lax.axis_index("c")
    base = wid * _UG_PER_W
    for k in range(_UG_N):
        pltpu.sync_copy(pos_hbm.at[pl.ds(base + k * _UG_CH, _UG_CH)],
                        idx_v.at[k])
    bufs = (rows_a, rows_b)
    sems = (sem_a, sem_b)
    handles = [None, None]
    handles[0] = pltpu.async_copy(y_hbm.at[idx_v.at[0]], bufs[0], sems[0])
    for k in range(_UG_N):
        if k + 1 < _UG_N:
            nxt = (k + 1) % 2
            handles[nxt] = pltpu.async_copy(
                y_hbm.at[idx_v.at[k + 1]], bufs[nxt], sems[nxt])
        handles[k % 2].wait()
        pltpu.sync_copy(bufs[k % 2],
                        out_hbm.at[pl.ds(base + k * _UG_CH, _UG_CH)])


def _ungather(y_sorted, pos):
    return pl.kernel(
        _ungather_body,
        out_type=jax.ShapeDtypeStruct((B, D), jnp.float32),
        mesh=_SC_MESH,
        scratch_types=[
            pltpu.VMEM((_UG_N, _UG_CH), jnp.int32),
            pltpu.VMEM((_UG_CH, D), jnp.float32),
            pltpu.VMEM((_UG_CH, D), jnp.float32),
            pltpu.SemaphoreType.DMA,
            pltpu.SemaphoreType.DMA,
        ],
    )(y_sorted, pos)


def _compose_d1_body(wr_ref, w1_ref, br_ref, b1_ref, t_ref, bt_ref):
    w1 = w1_ref[0]
    t_ref[0] = jnp.dot(wr_ref[...], w1, preferred_element_type=jnp.float32)
    bt_ref[0] = jnp.dot(br_ref[...], w1, preferred_element_type=jnp.float32) + b1_ref[0]


def _compose_d2_body(t_ref, w2_ref, bt_ref, b2_ref, wc_ref, bc_ref):
    w2 = w2_ref[0]
    wc_ref[0] = jnp.dot(t_ref[0], w2, preferred_element_type=jnp.float32)
    bc_ref[0] = jnp.dot(bt_ref[0], w2, preferred_element_type=jnp.float32) + b2_ref[0]


def _compose(W_root, b_root, W_d1, b_d1, W_d2, b_d2):
    br = b_root.reshape(1, D)
    b1 = b_d1.reshape(2, 1, D)
    b2 = b_d2.reshape(4, 1, D)
    T, bt = pl.pallas_call(
        _compose_d1_body,
        grid=(2,),
        in_specs=[
            pl.BlockSpec((D, D), lambda c: (0, 0)),
            pl.BlockSpec((1, D, D), lambda c: (c, 0, 0)),
            pl.BlockSpec((1, D), lambda c: (0, 0)),
            pl.BlockSpec((1, 1, D), lambda c: (c, 0, 0)),
        ],
        out_specs=[
            pl.BlockSpec((1, D, D), lambda c: (c, 0, 0)),
            pl.BlockSpec((1, 1, D), lambda c: (c, 0, 0)),
        ],
        out_shape=[
            jax.ShapeDtypeStruct((2, D, D), jnp.float32),
            jax.ShapeDtypeStruct((2, 1, D), jnp.float32),
        ],
    )(W_root, W_d1, br, b1)
    Wc, bc = pl.pallas_call(
        _compose_d2_body,
        grid=(4,),
        in_specs=[
            pl.BlockSpec((1, D, D), lambda e: (e // 2, 0, 0)),
            pl.BlockSpec((1, D, D), lambda e: (e, 0, 0)),
            pl.BlockSpec((1, 1, D), lambda e: (e // 2, 0, 0)),
            pl.BlockSpec((1, 1, D), lambda e: (e, 0, 0)),
        ],
        out_specs=[
            pl.BlockSpec((1, D, D), lambda e: (e, 0, 0)),
            pl.BlockSpec((1, 1, D), lambda e: (e, 0, 0)),
        ],
        out_shape=[
            jax.ShapeDtypeStruct((4, D, D), jnp.float32),
            jax.ShapeDtypeStruct((4, 1, D), jnp.float32),
        ],
    )(T, W_d2, bt, b2)
    return Wc, bc


def _routed_mm_body(off_ref, x_ref, wc_ref, bc_ref, o_ref):
    del off_ref
    o_ref[...] = (
        jnp.dot(x_ref[...], wc_ref[0], preferred_element_type=jnp.float32)
        + bc_ref[0]
    )


def _block_expert(b, off_ref):
    s = b * BLK
    return (
        (s >= off_ref[1]).astype(jnp.int32)
        + (s >= off_ref[2]).astype(jnp.int32)
        + (s >= off_ref[3]).astype(jnp.int32)
    )


def _routed_mm(off, x_sorted, Wc, bc):
    spec = pltpu.PrefetchScalarGridSpec(
        num_scalar_prefetch=1,
        grid=(NBLK,),
        in_specs=[
            pl.BlockSpec((BLK, D), lambda b, off_ref: (b, 0)),
            pl.BlockSpec((1, D, D), lambda b, off_ref: (_block_expert(b, off_ref), 0, 0)),
            pl.BlockSpec((1, 1, D), lambda b, off_ref: (_block_expert(b, off_ref), 0, 0)),
        ],
        out_specs=pl.BlockSpec((BLK, D), lambda b, off_ref: (b, 0)),
    )
    return pl.pallas_call(
        _routed_mm_body,
        grid_spec=spec,
        out_shape=jax.ShapeDtypeStruct((C, D), jnp.float32),
    )(off, x_sorted, Wc, bc)


def kernel(x, path_mask, W_root, b_root, W_d1, b_d1, W_d2, b_d2):
    leaf = path_mask[:, 0] * 2 + path_mask[:, 1]
    pos, src, offs = _route(leaf)
    x_sorted = _gatherx(x, src)
    Wc, bc = _compose(W_root, b_root, W_d1, b_d1, W_d2, b_d2)
    y_sorted = _routed_mm(offs, x_sorted, Wc, bc)
    return _ungather(y_sorted, pos)
